# Initial kernel scaffold; baseline (speedup 1.0000x reference)
import jax
import jax.numpy as jnp
from jax.experimental import pallas as pl


def kernel(x, edge_index, W0, b0, Ws0, bs0, bb0, W, b, Ws, bs, bb, Wl, bl):
    raise NotImplementedError("write your pallas kernel here")


# speedup vs baseline: 5.1706x; 5.1706x over previous
import functools

import jax
import jax.numpy as jnp
from jax import lax
from jax.experimental import pallas as pl
from jax.experimental.pallas import tpu as pltpu
from jax.experimental.pallas import tpu_sc as plsc

N = 10000
N_PAD = 10240
E = 320000
D = 128
H = 64
LAYERS = 6
OUT = 64

NC = 2
NS = 16
NW = NC * NS
EPW = E // NW
CH = 80
NCHUNK = EPW // CH
RPT = N_PAD // NS

ROWS_BLK = 1280
GRID = N_PAD // ROWS_BLK


def _sc_agg_body(p_hbm, src_hbm, dst_hbm, zeros_hbm, out_hbm,
                 src_v, dst_v, rows_v, agg_sh, sem):
    cid = lax.axis_index("c")
    sid = lax.axis_index("s")
    pltpu.sync_copy(zeros_hbm.at[pl.ds(sid * RPT, RPT)],
                    agg_sh.at[pl.ds(sid * RPT, RPT)])
    plsc.subcore_barrier()
    wid = cid * NS + sid

    def body(j, carry):
        pltpu.sync_copy(src_hbm.at[wid, j], src_v)
        pltpu.sync_copy(dst_hbm.at[wid, j], dst_v)
        pltpu.async_copy(p_hbm.at[src_v], rows_v, sem).wait()
        pltpu.sync_copy(rows_v, agg_sh.at[dst_v], add=True)
        return carry

    lax.fori_loop(0, NCHUNK, body, 0)
    plsc.subcore_barrier()
    pltpu.sync_copy(agg_sh.at[pl.ds(sid * RPT, RPT)],
                    out_hbm.at[cid, pl.ds(sid * RPT, RPT)])


_sc_agg = pl.kernel(
    _sc_agg_body,
    mesh=plsc.VectorSubcoreMesh(core_axis_name="c", subcore_axis_name="s"),
    compiler_params=pltpu.CompilerParams(use_tc_tiling_on_sc=False),
    out_type=jax.ShapeDtypeStruct((NC, N_PAD, H), jnp.float32),
    scratch_types=[
        pltpu.VMEM((CH,), jnp.int32),
        pltpu.VMEM((CH,), jnp.int32),
        pltpu.VMEM((CH, H), jnp.float32),
        pltpu.VMEM_SHARED((N_PAD, H), jnp.float32),
        pltpu.SemaphoreType.DMA,
    ],
)


def _tc0_body(x_ref, w_ref, ws_ref, bias_ref, p_ref, s_ref):
    xb = x_ref[...]
    p_ref[...] = jnp.dot(xb, w_ref[...], preferred_element_type=jnp.float32)
    s_ref[...] = (jnp.dot(xb, ws_ref[...], preferred_element_type=jnp.float32)
                  + bias_ref[...])


_tc0 = pl.pallas_call(
    _tc0_body,
    grid=(GRID,),
    in_specs=[
        pl.BlockSpec((ROWS_BLK, D), lambda i: (i, 0)),
        pl.BlockSpec((D, H), lambda i: (0, 0)),
        pl.BlockSpec((D, H), lambda i: (0, 0)),
        pl.BlockSpec((1, H), lambda i: (0, 0)),
    ],
    out_specs=[
        pl.BlockSpec((ROWS_BLK, H), lambda i: (i, 0)),
        pl.BlockSpec((ROWS_BLK, H), lambda i: (i, 0)),
    ],
    out_shape=[
        jax.ShapeDtypeStruct((N_PAD, H), jnp.float32),
        jax.ShapeDtypeStruct((N_PAD, H), jnp.float32),
    ],
)


def _tcstep_body(a0_ref, a1_ref, s_ref, w_ref, ws_ref, bias_ref, wl_ref,
                 acc_ref, p_out, s_out, acc_out):
    h = jnp.maximum(a0_ref[...] + a1_ref[...] + s_ref[...], 0.0)
    p_out[...] = jnp.dot(h, w_ref[...], preferred_element_type=jnp.float32)
    s_out[...] = (jnp.dot(h, ws_ref[...], preferred_element_type=jnp.float32)
                  + bias_ref[...])
    acc_out[...] = acc_ref[...] + jnp.dot(
        h, wl_ref[...], preferred_element_type=jnp.float32)


_tcstep = pl.pallas_call(
    _tcstep_body,
    grid=(GRID,),
    in_specs=[
        pl.BlockSpec((ROWS_BLK, H), lambda i: (i, 0)),
        pl.BlockSpec((ROWS_BLK, H), lambda i: (i, 0)),
        pl.BlockSpec((ROWS_BLK, H), lambda i: (i, 0)),
        pl.BlockSpec((H, H), lambda i: (0, 0)),
        pl.BlockSpec((H, H), lambda i: (0, 0)),
        pl.BlockSpec((1, H), lambda i: (0, 0)),
        pl.BlockSpec((H, OUT), lambda i: (0, 0)),
        pl.BlockSpec((ROWS_BLK, OUT), lambda i: (i, 0)),
    ],
    out_specs=[
        pl.BlockSpec((ROWS_BLK, H), lambda i: (i, 0)),
        pl.BlockSpec((ROWS_BLK, H), lambda i: (i, 0)),
        pl.BlockSpec((ROWS_BLK, OUT), lambda i: (i, 0)),
    ],
    out_shape=[
        jax.ShapeDtypeStruct((N_PAD, H), jnp.float32),
        jax.ShapeDtypeStruct((N_PAD, H), jnp.float32),
        jax.ShapeDtypeStruct((N_PAD, OUT), jnp.float32),
    ],
)


def _tcfin_body(a0_ref, a1_ref, s_ref, wl_ref, bl_ref, acc_ref, out_ref):
    h = jnp.maximum(a0_ref[...] + a1_ref[...] + s_ref[...], 0.0)
    out_ref[...] = (acc_ref[...] + bl_ref[...]
                    + jnp.dot(h, wl_ref[...],
                              preferred_element_type=jnp.float32))


_tcfin = pl.pallas_call(
    _tcfin_body,
    grid=(GRID,),
    in_specs=[
        pl.BlockSpec((ROWS_BLK, H), lambda i: (i, 0)),
        pl.BlockSpec((ROWS_BLK, H), lambda i: (i, 0)),
        pl.BlockSpec((ROWS_BLK, H), lambda i: (i, 0)),
        pl.BlockSpec((H, OUT), lambda i: (0, 0)),
        pl.BlockSpec((1, OUT), lambda i: (0, 0)),
        pl.BlockSpec((ROWS_BLK, OUT), lambda i: (i, 0)),
    ],
    out_specs=pl.BlockSpec((ROWS_BLK, OUT), lambda i: (i, 0)),
    out_shape=jax.ShapeDtypeStruct((N_PAD, OUT), jnp.float32),
)


def kernel(x, edge_index, W0, b0, Ws0, bs0, bb0, W, b, Ws, bs, bb, Wl, bl):
    src = edge_index[0].reshape(NW, NCHUNK, CH)
    dst = edge_index[1].reshape(NW, NCHUNK, CH)
    zeros_nh = jnp.zeros((N_PAD, H), jnp.float32)
    xp = jnp.pad(x, ((0, N_PAD - N), (0, 0)))

    bias0 = (b0 + bs0 + bb0).reshape(1, H)
    p, s = _tc0(xp, W0, Ws0, bias0)
    acc = jnp.zeros((N_PAD, OUT), jnp.float32)
    for i in range(LAYERS - 1):
        agg = _sc_agg(p, src, dst, zeros_nh)
        bias_i = (b[i] + bs[i] + bb[i]).reshape(1, H)
        p, s, acc = _tcstep(agg[0], agg[1], s, W[i], Ws[i], bias_i,
                            Wl[i * H:(i + 1) * H], acc)
    agg = _sc_agg(p, src, dst, zeros_nh)
    out = _tcfin(agg[0], agg[1], s, Wl[(LAYERS - 1) * H:], bl.reshape(1, OUT),
                 acc)
    return out[:N]


# speedup vs baseline: 15.5987x; 3.0168x over previous
import functools

import jax
import jax.numpy as jnp
from jax import lax
from jax.experimental import pallas as pl
from jax.experimental.pallas import tpu as pltpu
from jax.experimental.pallas import tpu_sc as plsc

N = 10000
N_PAD = 10240
E = 320000
D = 128
H = 64
LAYERS = 6
OUT = 64

NC = 2
NS = 16
NW = NC * NS
CH = 128
NCHUNK = 80
EPW = NCHUNK * CH
E_PAD = NW * EPW
NBUF = 4
RPT = N_PAD // NS

ROWS_BLK = 1280
GRID = N_PAD // ROWS_BLK


def _sc_agg_body(p_hbm, src_hbm, dst_hbm, zeros_hbm, out_hbm,
                 srcs_v, dsts_v, bufs_v, agg_sh,
                 gsem0, gsem1, gsem2, gsem3):
    cid = lax.axis_index("c")
    sid = lax.axis_index("s")
    gsems = [gsem0, gsem1, gsem2, gsem3]
    pltpu.sync_copy(zeros_hbm.at[pl.ds(sid * RPT, RPT)],
                    agg_sh.at[pl.ds(sid * RPT, RPT)])
    wid = cid * NS + sid
    pltpu.sync_copy(src_hbm.at[wid], srcs_v)
    pltpu.sync_copy(dst_hbm.at[wid], dsts_v)
    plsc.subcore_barrier()

    for b in range(NBUF - 1):
        pltpu.async_copy(p_hbm.at[srcs_v.at[b]], bufs_v.at[b], gsems[b])

    def outer(jj, carry):
        for b in range(NBUF):
            j = jj * NBUF + b
            bn = (b + NBUF - 1) % NBUF

            @pl.when(j + NBUF - 1 < NCHUNK)
            def _():
                pltpu.async_copy(p_hbm.at[srcs_v.at[j + NBUF - 1]],
                                 bufs_v.at[bn], gsems[bn])

            pltpu.make_async_copy(p_hbm.at[srcs_v.at[j]], bufs_v.at[b],
                                  gsems[b]).wait()
            pltpu.sync_copy(bufs_v.at[b], agg_sh.at[dsts_v.at[j]], add=True)
        return carry

    lax.fori_loop(0, NCHUNK // NBUF, outer, 0)
    plsc.subcore_barrier()
    pltpu.sync_copy(agg_sh.at[pl.ds(sid * RPT, RPT)],
                    out_hbm.at[cid, pl.ds(sid * RPT, RPT)])


_sc_agg = pl.kernel(
    _sc_agg_body,
    mesh=plsc.VectorSubcoreMesh(core_axis_name="c", subcore_axis_name="s"),
    compiler_params=pltpu.CompilerParams(use_tc_tiling_on_sc=False),
    out_type=jax.ShapeDtypeStruct((NC, N_PAD, H), jnp.float32),
    scratch_types=[
        pltpu.VMEM((NCHUNK, CH), jnp.int32),
        pltpu.VMEM((NCHUNK, CH), jnp.int32),
        pltpu.VMEM((NBUF, CH, H), jnp.float32),
        pltpu.VMEM_SHARED((N_PAD, H), jnp.float32),
        pltpu.SemaphoreType.DMA,
        pltpu.SemaphoreType.DMA,
        pltpu.SemaphoreType.DMA,
        pltpu.SemaphoreType.DMA,
    ],
)


def _tc0_body(x_ref, w_ref, ws_ref, bias_ref, p_ref, s_ref):
    xb = x_ref[...]
    p_ref[...] = jnp.dot(xb, w_ref[...], preferred_element_type=jnp.float32)
    s_ref[...] = (jnp.dot(xb, ws_ref[...], preferred_element_type=jnp.float32)
                  + bias_ref[...])


_tc0 = pl.pallas_call(
    _tc0_body,
    grid=(GRID,),
    in_specs=[
        pl.BlockSpec((ROWS_BLK, D), lambda i: (i, 0)),
        pl.BlockSpec((D, H), lambda i: (0, 0)),
        pl.BlockSpec((D, H), lambda i: (0, 0)),
        pl.BlockSpec((1, H), lambda i: (0, 0)),
    ],
    out_specs=[
        pl.BlockSpec((ROWS_BLK, H), lambda i: (i, 0)),
        pl.BlockSpec((ROWS_BLK, H), lambda i: (i, 0)),
    ],
    out_shape=[
        jax.ShapeDtypeStruct((N_PAD, H), jnp.float32),
        jax.ShapeDtypeStruct((N_PAD, H), jnp.float32),
    ],
)


def _tcstep_body(a0_ref, a1_ref, s_ref, w_ref, ws_ref, bias_ref, wl_ref,
                 acc_ref, p_out, s_out, acc_out):
    h = jnp.maximum(a0_ref[...] + a1_ref[...] + s_ref[...], 0.0)
    p_out[...] = jnp.dot(h, w_ref[...], preferred_element_type=jnp.float32)
    s_out[...] = (jnp.dot(h, ws_ref[...], preferred_element_type=jnp.float32)
                  + bias_ref[...])
    acc_out[...] = acc_ref[...] + jnp.dot(
        h, wl_ref[...], preferred_element_type=jnp.float32)


_tcstep = pl.pallas_call(
    _tcstep_body,
    grid=(GRID,),
    in_specs=[
        pl.BlockSpec((ROWS_BLK, H), lambda i: (i, 0)),
        pl.BlockSpec((ROWS_BLK, H), lambda i: (i, 0)),
        pl.BlockSpec((ROWS_BLK, H), lambda i: (i, 0)),
        pl.BlockSpec((H, H), lambda i: (0, 0)),
        pl.BlockSpec((H, H), lambda i: (0, 0)),
        pl.BlockSpec((1, H), lambda i: (0, 0)),
        pl.BlockSpec((H, OUT), lambda i: (0, 0)),
        pl.BlockSpec((ROWS_BLK, OUT), lambda i: (i, 0)),
    ],
    out_specs=[
        pl.BlockSpec((ROWS_BLK, H), lambda i: (i, 0)),
        pl.BlockSpec((ROWS_BLK, H), lambda i: (i, 0)),
        pl.BlockSpec((ROWS_BLK, OUT), lambda i: (i, 0)),
    ],
    out_shape=[
        jax.ShapeDtypeStruct((N_PAD, H), jnp.float32),
        jax.ShapeDtypeStruct((N_PAD, H), jnp.float32),
        jax.ShapeDtypeStruct((N_PAD, OUT), jnp.float32),
    ],
)


def _tcfin_body(a0_ref, a1_ref, s_ref, wl_ref, bl_ref, acc_ref, out_ref):
    h = jnp.maximum(a0_ref[...] + a1_ref[...] + s_ref[...], 0.0)
    out_ref[...] = (acc_ref[...] + bl_ref[...]
                    + jnp.dot(h, wl_ref[...],
                              preferred_element_type=jnp.float32))


_tcfin = pl.pallas_call(
    _tcfin_body,
    grid=(GRID,),
    in_specs=[
        pl.BlockSpec((ROWS_BLK, H), lambda i: (i, 0)),
        pl.BlockSpec((ROWS_BLK, H), lambda i: (i, 0)),
        pl.BlockSpec((ROWS_BLK, H), lambda i: (i, 0)),
        pl.BlockSpec((H, OUT), lambda i: (0, 0)),
        pl.BlockSpec((1, OUT), lambda i: (0, 0)),
        pl.BlockSpec((ROWS_BLK, OUT), lambda i: (i, 0)),
    ],
    out_specs=pl.BlockSpec((ROWS_BLK, OUT), lambda i: (i, 0)),
    out_shape=jax.ShapeDtypeStruct((N_PAD, OUT), jnp.float32),
)


def kernel(x, edge_index, W0, b0, Ws0, bs0, bb0, W, b, Ws, bs, bb, Wl, bl):
    n_extra = E_PAD - E
    pad_src = (jnp.arange(n_extra, dtype=jnp.int32) * 37) % N
    pad_dst = N + (jnp.arange(n_extra, dtype=jnp.int32) % (N_PAD - N))
    src = jnp.concatenate([edge_index[0], pad_src]).reshape(NW, NCHUNK, CH)
    dst = jnp.concatenate([edge_index[1], pad_dst]).reshape(NW, NCHUNK, CH)
    zeros_nh = jnp.zeros((N_PAD, H), jnp.float32)
    xp = jnp.pad(x, ((0, N_PAD - N), (0, 0)))

    bias0 = (b0 + bs0 + bb0).reshape(1, H)
    p, s = _tc0(xp, W0, Ws0, bias0)
    acc = jnp.zeros((N_PAD, OUT), jnp.float32)
    for i in range(LAYERS - 1):
        agg = _sc_agg(p, src, dst, zeros_nh)
        bias_i = (b[i] + bs[i] + bb[i]).reshape(1, H)
        p, s, acc = _tcstep(agg[0], agg[1], s, W[i], Ws[i], bias_i,
                            Wl[i * H:(i + 1) * H], acc)
    agg = _sc_agg(p, src, dst, zeros_nh)
    out = _tcfin(agg[0], agg[1], s, Wl[(LAYERS - 1) * H:], bl.reshape(1, OUT),
                 acc)
    return out[:N]
